# baseline (speedup 1.0000x reference)
import functools

import jax
import jax.numpy as jnp
from jax import lax
from jax.experimental import pallas as pl
from jax.experimental.pallas import tpu as pltpu
from jax.experimental.pallas import tpu_sc as plsc

N_NODES = 10000
N_EDGES = 320000
D = 128

NC = 2
NS = 16
NW = NC * NS
N_ACC = 10240
RPT = N_ACC // NS
ZR = 128

L1_PARTS = ((6400, 80), (3600, 120))
L2_PARTS = ((4000, 80), (3200, 80), (1600, 80), (1200, 120))


def _sc_segment_body(epw, ch, base0, x_hbm, src_hbm, dst_hbm, a_hbm,
                     out_hbm, acc_sh, src_v, dst_v, a_v, rows_v,
                     zero_v, sem):
    nchunk = epw // ch
    cid = lax.axis_index("c")
    sid = lax.axis_index("s")
    wid = cid * NS + sid

    def _zero_row(i, _):
        for j in range(D // 16):
            zero_v[i, pl.ds(j * 16, 16)] = jnp.zeros((16,), jnp.float32)
        return 0
    lax.fori_loop(0, ZR, _zero_row, 0)
    for r in range(RPT // ZR):
        pltpu.sync_copy(zero_v, acc_sh.at[pl.ds(sid * RPT + r * ZR, ZR)])
    plsc.subcore_barrier()

    def _chunk(ci, _):
        loc = wid * epw + ci * ch
        gbl = base0 + loc
        pltpu.sync_copy(src_hbm.at[pl.ds(gbl, ch)], src_v)
        gat = pltpu.async_copy(x_hbm.at[src_v], rows_v, sem)
        pltpu.sync_copy(a_hbm.at[pl.ds(loc, ch)], a_v)
        pltpu.sync_copy(dst_hbm.at[pl.ds(gbl, ch)], dst_v)
        gat.wait()

        def _edge(i, _):
            for j in range(D // 16):
                sl = pl.ds(j * 16, 16)
                rows_v[i, sl] = rows_v[i, sl] * a_v[i, sl]
            return 0
        lax.fori_loop(0, ch, _edge, 0)

        pltpu.sync_copy(rows_v, acc_sh.at[dst_v], add=True)
        return 0

    lax.fori_loop(0, nchunk, _chunk, 0)
    plsc.subcore_barrier()

    pltpu.sync_copy(acc_sh.at[pl.ds(sid * RPT, RPT)],
                    out_hbm.at[cid, pl.ds(sid * RPT, RPT)])


def _sc_segment(x, src, dst, a, epw, ch, base0):
    mesh = plsc.VectorSubcoreMesh(core_axis_name="c", subcore_axis_name="s",
                                  num_cores=NC, num_subcores=NS)
    body = functools.partial(_sc_segment_body, epw, ch, base0)
    f = pl.kernel(
        body,
        out_type=jax.ShapeDtypeStruct((NC, N_ACC, D), jnp.float32),
        mesh=mesh,
        scratch_types=[
            pltpu.VMEM_SHARED((N_ACC, D), jnp.float32),
            pltpu.VMEM((ch,), jnp.int32),
            pltpu.VMEM((ch,), jnp.int32),
            pltpu.VMEM((ch, D), jnp.float32),
            pltpu.VMEM((ch, D), jnp.float32),
            pltpu.VMEM((ZR, D), jnp.float32),
            pltpu.SemaphoreType.DMA,
        ],
    )
    return f(x, src, dst, a)


_R0 = (13, 15, 26, 6)
_R1 = (17, 29, 16, 24)
_M32 = 0xFFFFFFFF


def _rng_body(k1, k2, row0, bm, scale_ref, o_ref):
    i = pl.program_id(0)
    d = D
    base = jnp.uint32((row0 * d) & _M32) + (i * bm * d).astype(jnp.uint32)
    n = (base
         + lax.broadcasted_iota(jnp.uint32, (bm, d), 0) * jnp.uint32(d)
         + lax.broadcasted_iota(jnp.uint32, (bm, d), 1))
    ks = (k1, k2, k1 ^ k2 ^ 0x1BD11BDA)
    x0 = jnp.full((bm, d), jnp.uint32(ks[0]), jnp.uint32)
    x1 = n + jnp.uint32(ks[1])
    for r, rots in enumerate((_R0, _R1, _R0, _R1, _R0)):
        for rot in rots:
            x0 = x0 + x1
            x1 = (x1 << jnp.uint32(rot)) | (x1 >> jnp.uint32(32 - rot))
            x1 = x0 ^ x1
        x0 = x0 + jnp.uint32(ks[(r + 1) % 3])
        x1 = x1 + jnp.uint32((ks[(r + 2) % 3] + r + 1) & _M32)
    bits = x0 ^ x1
    g = (bits >> jnp.uint32(9)) | jnp.uint32(0x3F800000)
    f = lax.bitcast_convert_type(g, jnp.float32) - 1.0
    lo = jnp.float32(-0.99999994)
    u = jnp.maximum(lo, f * (1.0 - lo) + lo)
    w = -jnp.log((1.0 - u) * (1.0 + u))
    wl = w - 2.5
    p1 = jnp.float32(2.81022636e-08)
    for c in (3.43273939e-07, -3.5233877e-06, -4.39150654e-06, 0.00021858087,
              -0.00125372503, -0.00417768164, 0.246640727, 1.50140941):
        p1 = jnp.float32(c) + p1 * wl
    ws = jnp.sqrt(w) - 3.0
    p2 = jnp.float32(-0.000200214257)
    for c in (0.000100950558, 0.00134934322, -0.00367342844, 0.00573950773,
              -0.0076224613, 0.00943887047, 1.00167406, 2.83297682):
        p2 = jnp.float32(c) + p2 * ws
    eps = jnp.float32(1.4142135381698608) * jnp.where(w < 5.0, p1, p2) * u
    o_ref[...] = scale_ref[0:1, :] + scale_ref[1:2, :] * eps


def _rng_scale(k1, k2, scale, row0, nrows):
    bm = 12800
    body = functools.partial(_rng_body, k1, k2, row0, bm)
    return pl.pallas_call(
        body,
        grid=(nrows // bm,),
        in_specs=[pl.BlockSpec((2, D), lambda i: (0, 0))],
        out_specs=pl.BlockSpec((bm, D), lambda i: (i, 0)),
        out_shape=jax.ShapeDtypeStruct((nrows, D), jnp.float32),
    )(scale)


def _mm_relu_body(*refs):
    o_ref = refs[-1]
    w_ref, b_ref = refs[-3], refs[-2]
    s = refs[0][0] + refs[0][1]
    for p in refs[1:-3]:
        s = s + p[0] + p[1]
    y = jnp.dot(s, w_ref[...], preferred_element_type=jnp.float32)
    o_ref[...] = jnp.maximum(y + b_ref[...], 0.0)


def _mm_softmax_body(*refs):
    o_ref = refs[-1]
    w_ref, b_ref = refs[-3], refs[-2]
    s = refs[0][0] + refs[0][1]
    for p in refs[1:-3]:
        s = s + p[0] + p[1]
    z = jnp.dot(s, w_ref[...], preferred_element_type=jnp.float32)
    z = z + b_ref[...]
    m = jnp.max(z, axis=-1, keepdims=True)
    e = jnp.exp(z - m)
    o_ref[...] = e / jnp.sum(e, axis=-1, keepdims=True)


def _tc_dense(body, parts, w, b, bm=640):
    grid = (N_ACC // bm,)
    in_specs = ([pl.BlockSpec((NC, bm, D), lambda i: (0, i, 0))
                 for _ in parts]
                + [pl.BlockSpec((D, D), lambda i: (0, 0)),
                   pl.BlockSpec((1, D), lambda i: (0, 0))])
    return pl.pallas_call(
        body,
        grid=grid,
        in_specs=in_specs,
        out_specs=pl.BlockSpec((bm, D), lambda i: (i, 0)),
        out_shape=jax.ShapeDtypeStruct((N_ACC, D), jnp.float32),
    )(*parts, w, b)


def kernel(x, edge_index, W0, b0, W1, b1, a_mu_0, a_log_sigma_0,
           a_mu_1, a_log_sigma_1):
    src = edge_index[0]
    dst = edge_index[1]

    K0 = (1832780943, 270669613)
    K1 = (64467757, 2916123636)

    scale0 = jnp.stack([a_mu_0, jnp.exp(a_log_sigma_0)])
    scale1 = jnp.stack([a_mu_1, jnp.exp(a_log_sigma_1)])

    parts0 = []
    row = 0
    for epw, ch in L1_PARTS:
        nrows = epw * NW
        a_part = _rng_scale(K0[0], K0[1], scale0, row, nrows)
        parts0.append(_sc_segment(x, src, dst, a_part, epw, ch, row))
        row += nrows

    a1_parts = []
    row = 0
    for epw, ch in L2_PARTS:
        nrows = epw * NW
        a1_parts.append(_rng_scale(K1[0], K1[1], scale1, row, nrows))
        row += nrows

    h = _tc_dense(_mm_relu_body, parts0, W0, b0.reshape(1, D))

    parts1 = []
    row = 0
    for (epw, ch), a_part in zip(L2_PARTS, a1_parts):
        parts1.append(_sc_segment(h, src, dst, a_part, epw, ch, row))
        row += epw * NW

    n_out = W1.shape[1]
    W1p = jnp.zeros((D, D), jnp.float32).at[:, :n_out].set(W1)
    b1p = jnp.full((1, D), -1e30, jnp.float32).at[0, :n_out].set(b1)
    out = _tc_dense(_mm_softmax_body, parts1, W1p, b1p)
    return out[:N_NODES, :n_out]


# speedup vs baseline: 1.2740x; 1.2740x over previous
import functools

import jax
import jax.numpy as jnp
from jax import lax
from jax.experimental import pallas as pl
from jax.experimental.pallas import tpu as pltpu
from jax.experimental.pallas import tpu_sc as plsc

N_NODES = 10000
N_EDGES = 320000
D = 128

NC = 2
NS = 16
NW = NC * NS
N_ACC = 10240
RPT = N_ACC // NS
ZR = 128

L1_PARTS = ((6400, 80), (3600, 120))
L2_PARTS = ((4000, 80), (3200, 80), (1600, 80), (1200, 120))


def _sc_segment_body(epw, ch, base0, x_hbm, src_hbm, dst_hbm, a_hbm,
                     out_hbm, acc_sh, src_v, dst_v, a_v, rows_v,
                     zero_v, sem):
    nchunk = epw // ch
    cid = lax.axis_index("c")
    sid = lax.axis_index("s")
    wid = cid * NS + sid

    def _zero_row(i, _):
        for j in range(D // 16):
            zero_v[i, pl.ds(j * 16, 16)] = jnp.zeros((16,), jnp.float32)
        return 0
    lax.fori_loop(0, ZR, _zero_row, 0)
    for r in range(RPT // ZR):
        pltpu.sync_copy(zero_v, acc_sh.at[pl.ds(sid * RPT + r * ZR, ZR)])
    plsc.subcore_barrier()

    def _chunk(ci, _):
        loc = wid * epw + ci * ch
        gbl = base0 + loc
        pltpu.sync_copy(src_hbm.at[pl.ds(gbl, ch)], src_v)
        gat = pltpu.async_copy(x_hbm.at[src_v], rows_v, sem)
        pltpu.sync_copy(a_hbm.at[pl.ds(loc, ch)], a_v)
        pltpu.sync_copy(dst_hbm.at[pl.ds(gbl, ch)], dst_v)
        gat.wait()

        def _edge(i, _):
            for j in range(D // 16):
                sl = pl.ds(j * 16, 16)
                rows_v[i, sl] = rows_v[i, sl] * a_v[i, sl]
            return 0
        lax.fori_loop(0, ch, _edge, 0)

        pltpu.sync_copy(rows_v, acc_sh.at[dst_v], add=True)
        return 0

    lax.fori_loop(0, nchunk, _chunk, 0)
    plsc.subcore_barrier()

    pltpu.sync_copy(acc_sh.at[pl.ds(sid * RPT, RPT)],
                    out_hbm.at[cid, pl.ds(sid * RPT, RPT)])


def _sc_segment(x, src, dst, a, epw, ch, base0):
    mesh = plsc.VectorSubcoreMesh(core_axis_name="c", subcore_axis_name="s",
                                  num_cores=NC, num_subcores=NS)
    body = functools.partial(_sc_segment_body, epw, ch, base0)
    f = pl.kernel(
        body,
        out_type=jax.ShapeDtypeStruct((NC, N_ACC, D), jnp.float32),
        mesh=mesh,
        scratch_types=[
            pltpu.VMEM_SHARED((N_ACC, D), jnp.float32),
            pltpu.VMEM((ch,), jnp.int32),
            pltpu.VMEM((ch,), jnp.int32),
            pltpu.VMEM((ch, D), jnp.float32),
            pltpu.VMEM((ch, D), jnp.float32),
            pltpu.VMEM((ZR, D), jnp.float32),
            pltpu.SemaphoreType.DMA,
        ],
    )
    return f(x, src, dst, a)


_R0 = (13, 15, 26, 6)
_R1 = (17, 29, 16, 24)
_M32 = 0xFFFFFFFF


def _rng_body(k1, k2, row0, bm, scale_ref, o_ref):
    i = pl.program_id(0)
    d = D
    base = jnp.uint32((row0 * d) & _M32) + (i * bm * d).astype(jnp.uint32)
    n = (base
         + lax.broadcasted_iota(jnp.uint32, (bm, d), 0) * jnp.uint32(d)
         + lax.broadcasted_iota(jnp.uint32, (bm, d), 1))
    ks = (k1, k2, k1 ^ k2 ^ 0x1BD11BDA)
    x0 = jnp.full((bm, d), jnp.uint32(ks[0]), jnp.uint32)
    x1 = n + jnp.uint32(ks[1])
    for r, rots in enumerate((_R0, _R1, _R0, _R1, _R0)):
        for rot in rots:
            x0 = x0 + x1
            x1 = (x1 << jnp.uint32(rot)) | (x1 >> jnp.uint32(32 - rot))
            x1 = x0 ^ x1
        x0 = x0 + jnp.uint32(ks[(r + 1) % 3])
        x1 = x1 + jnp.uint32((ks[(r + 2) % 3] + r + 1) & _M32)
    bits = x0 ^ x1
    g = (bits >> jnp.uint32(9)) | jnp.uint32(0x3F800000)
    f = lax.bitcast_convert_type(g, jnp.float32) - 1.0
    lo = jnp.float32(-0.99999994)
    u = jnp.maximum(lo, f * (1.0 - lo) + lo)
    w = -jnp.log((1.0 - u) * (1.0 + u))
    wl = w - 2.5
    p1 = jnp.float32(2.81022636e-08)
    for c in (3.43273939e-07, -3.5233877e-06, -4.39150654e-06, 0.00021858087,
              -0.00125372503, -0.00417768164, 0.246640727, 1.50140941):
        p1 = jnp.float32(c) + p1 * wl
    ws = jnp.sqrt(w) - 3.0
    p2 = jnp.float32(-0.000200214257)
    for c in (0.000100950558, 0.00134934322, -0.00367342844, 0.00573950773,
              -0.0076224613, 0.00943887047, 1.00167406, 2.83297682):
        p2 = jnp.float32(c) + p2 * ws
    eps = jnp.float32(1.4142135381698608) * jnp.where(w < 5.0, p1, p2) * u
    o_ref[...] = scale_ref[0:1, :] + scale_ref[1:2, :] * eps


def _rng_scale(k1, k2, scale, row0, nrows):
    bm = 6400
    body = functools.partial(_rng_body, k1, k2, row0, bm)
    return pl.pallas_call(
        body,
        grid=(nrows // bm,),
        in_specs=[pl.BlockSpec((2, D), lambda i: (0, 0))],
        out_specs=pl.BlockSpec((bm, D), lambda i: (i, 0)),
        out_shape=jax.ShapeDtypeStruct((nrows, D), jnp.float32),
    )(scale)


def _mm_relu_body(*refs):
    o_ref = refs[-1]
    w_ref, b_ref = refs[-3], refs[-2]
    s = refs[0][0] + refs[0][1]
    for p in refs[1:-3]:
        s = s + p[0] + p[1]
    y = jnp.dot(s, w_ref[...], preferred_element_type=jnp.float32)
    o_ref[...] = jnp.maximum(y + b_ref[...], 0.0)


def _mm_softmax_body(*refs):
    o_ref = refs[-1]
    w_ref, b_ref = refs[-3], refs[-2]
    s = refs[0][0] + refs[0][1]
    for p in refs[1:-3]:
        s = s + p[0] + p[1]
    z = jnp.dot(s, w_ref[...], preferred_element_type=jnp.float32)
    z = z + b_ref[...]
    m = jnp.max(z, axis=-1, keepdims=True)
    e = jnp.exp(z - m)
    o_ref[...] = e / jnp.sum(e, axis=-1, keepdims=True)


def _tc_dense(body, parts, w, b, bm=640):
    grid = (N_ACC // bm,)
    in_specs = ([pl.BlockSpec((NC, bm, D), lambda i: (0, i, 0))
                 for _ in parts]
                + [pl.BlockSpec((D, D), lambda i: (0, 0)),
                   pl.BlockSpec((1, D), lambda i: (0, 0))])
    return pl.pallas_call(
        body,
        grid=grid,
        in_specs=in_specs,
        out_specs=pl.BlockSpec((bm, D), lambda i: (i, 0)),
        out_shape=jax.ShapeDtypeStruct((N_ACC, D), jnp.float32),
    )(*parts, w, b)


def kernel(x, edge_index, W0, b0, W1, b1, a_mu_0, a_log_sigma_0,
           a_mu_1, a_log_sigma_1):
    src = edge_index[0]
    dst = edge_index[1]

    K0 = (1832780943, 270669613)
    K1 = (64467757, 2916123636)

    scale0 = jnp.stack([a_mu_0, jnp.exp(a_log_sigma_0)])
    scale1 = jnp.stack([a_mu_1, jnp.exp(a_log_sigma_1)])

    parts0 = []
    row = 0
    for epw, ch in L1_PARTS:
        nrows = epw * NW
        a_part = _rng_scale(K0[0], K0[1], scale0, row, nrows)
        parts0.append(_sc_segment(x, src, dst, a_part, epw, ch, row))
        row += nrows

    a1_parts = []
    row = 0
    for epw, ch in L2_PARTS:
        nrows = epw * NW
        a1_parts.append(_rng_scale(K1[0], K1[1], scale1, row, nrows))
        row += nrows

    h = _tc_dense(_mm_relu_body, parts0, W0, b0.reshape(1, D))

    parts1 = []
    row = 0
    for (epw, ch), a_part in zip(L2_PARTS, a1_parts):
        parts1.append(_sc_segment(h, src, dst, a_part, epw, ch, row))
        row += epw * NW

    n_out = W1.shape[1]
    W1p = jnp.zeros((D, D), jnp.float32).at[:, :n_out].set(W1)
    b1p = jnp.full((1, D), -1e30, jnp.float32).at[0, :n_out].set(b1)
    out = _tc_dense(_mm_softmax_body, parts1, W1p, b1p)
    return out[:N_NODES, :n_out]


# speedup vs baseline: 1.2774x; 1.0027x over previous
import functools

import jax
import jax.numpy as jnp
from jax import lax
from jax.experimental import pallas as pl
from jax.experimental.pallas import tpu as pltpu
from jax.experimental.pallas import tpu_sc as plsc

N_NODES = 10000
N_EDGES = 320000
D = 128

NC = 2
NS = 16
NW = NC * NS
N_ACC = 10240
RPT = N_ACC // NS
ZR = 128

L1_PARTS = ((6400, 80), (3600, 120))
L2_PARTS = ((4000, 80), (3200, 80), (2000, 80), (800, 80))


def _sc_segment_body(epw, ch, base0, x_hbm, src_hbm, dst_hbm, a_hbm,
                     out_hbm, acc_sh, src_v, dst_v, a_v, rows_v,
                     zero_v, sem):
    nchunk = epw // ch
    cid = lax.axis_index("c")
    sid = lax.axis_index("s")
    wid = cid * NS + sid

    def _zero_row(i, _):
        for j in range(D // 16):
            zero_v[i, pl.ds(j * 16, 16)] = jnp.zeros((16,), jnp.float32)
        return 0
    lax.fori_loop(0, ZR, _zero_row, 0)
    for r in range(RPT // ZR):
        pltpu.sync_copy(zero_v, acc_sh.at[pl.ds(sid * RPT + r * ZR, ZR)])
    plsc.subcore_barrier()

    def _chunk(ci, _):
        loc = wid * epw + ci * ch
        gbl = base0 + loc
        pltpu.sync_copy(src_hbm.at[pl.ds(gbl, ch)], src_v)
        gat = pltpu.async_copy(x_hbm.at[src_v], rows_v, sem)
        pltpu.sync_copy(a_hbm.at[pl.ds(loc, ch)], a_v)
        pltpu.sync_copy(dst_hbm.at[pl.ds(gbl, ch)], dst_v)
        gat.wait()

        def _edge(i, _):
            for j in range(D // 16):
                sl = pl.ds(j * 16, 16)
                rows_v[i, sl] = rows_v[i, sl] * a_v[i, sl]
            return 0
        lax.fori_loop(0, ch, _edge, 0)

        pltpu.sync_copy(rows_v, acc_sh.at[dst_v], add=True)
        return 0

    lax.fori_loop(0, nchunk, _chunk, 0)
    plsc.subcore_barrier()

    pltpu.sync_copy(acc_sh.at[pl.ds(sid * RPT, RPT)],
                    out_hbm.at[cid, pl.ds(sid * RPT, RPT)])


def _sc_segment(x, src, dst, a, epw, ch, base0):
    mesh = plsc.VectorSubcoreMesh(core_axis_name="c", subcore_axis_name="s",
                                  num_cores=NC, num_subcores=NS)
    body = functools.partial(_sc_segment_body, epw, ch, base0)
    f = pl.kernel(
        body,
        out_type=jax.ShapeDtypeStruct((NC, N_ACC, D), jnp.float32),
        mesh=mesh,
        scratch_types=[
            pltpu.VMEM_SHARED((N_ACC, D), jnp.float32),
            pltpu.VMEM((ch,), jnp.int32),
            pltpu.VMEM((ch,), jnp.int32),
            pltpu.VMEM((ch, D), jnp.float32),
            pltpu.VMEM((ch, D), jnp.float32),
            pltpu.VMEM((ZR, D), jnp.float32),
            pltpu.SemaphoreType.DMA,
        ],
    )
    return f(x, src, dst, a)


_R0 = (13, 15, 26, 6)
_R1 = (17, 29, 16, 24)
_M32 = 0xFFFFFFFF


def _rng_body(k1, k2, row0, bm, scale_ref, o_ref):
    i = pl.program_id(0)
    d = D
    base = jnp.uint32((row0 * d) & _M32) + (i * bm * d).astype(jnp.uint32)
    n = (base
         + lax.broadcasted_iota(jnp.uint32, (bm, d), 0) * jnp.uint32(d)
         + lax.broadcasted_iota(jnp.uint32, (bm, d), 1))
    ks = (k1, k2, k1 ^ k2 ^ 0x1BD11BDA)
    x0 = jnp.full((bm, d), jnp.uint32(ks[0]), jnp.uint32)
    x1 = n + jnp.uint32(ks[1])
    for r, rots in enumerate((_R0, _R1, _R0, _R1, _R0)):
        for rot in rots:
            x0 = x0 + x1
            x1 = (x1 << jnp.uint32(rot)) | (x1 >> jnp.uint32(32 - rot))
            x1 = x0 ^ x1
        x0 = x0 + jnp.uint32(ks[(r + 1) % 3])
        x1 = x1 + jnp.uint32((ks[(r + 2) % 3] + r + 1) & _M32)
    bits = x0 ^ x1
    g = (bits >> jnp.uint32(9)) | jnp.uint32(0x3F800000)
    f = lax.bitcast_convert_type(g, jnp.float32)
    lo = jnp.float32(-0.99999994)
    sp = jnp.float32(1.99999994)
    u = jnp.maximum(lo, f * sp + jnp.float32(lo - 1.99999994))
    w = jnp.log2((1.0 - u) * (1.0 + u)) * jnp.float32(-0.6931471805599453)
    wl = w - 2.5
    p1 = jnp.float32(2.81022636e-08)
    for c in (3.43273939e-07, -3.5233877e-06, -4.39150654e-06, 0.00021858087,
              -0.00125372503, -0.00417768164, 0.246640727, 1.50140941):
        p1 = jnp.float32(c) + p1 * wl
    ws = jnp.sqrt(w) - 3.0
    p2 = jnp.float32(-0.000200214257)
    for c in (0.000100950558, 0.00134934322, -0.00367342844, 0.00573950773,
              -0.0076224613, 0.00943887047, 1.00167406, 2.83297682):
        p2 = jnp.float32(c) + p2 * ws
    eps = jnp.where(w < 5.0, p1, p2) * u
    o_ref[...] = scale_ref[0:1, :] + scale_ref[1:2, :] * eps


def _rng_scale(k1, k2, scale, row0, nrows):
    bm = 6400
    body = functools.partial(_rng_body, k1, k2, row0, bm)
    return pl.pallas_call(
        body,
        grid=(nrows // bm,),
        in_specs=[pl.BlockSpec((2, D), lambda i: (0, 0))],
        out_specs=pl.BlockSpec((bm, D), lambda i: (i, 0)),
        out_shape=jax.ShapeDtypeStruct((nrows, D), jnp.float32),
    )(scale)


def _mm_relu_body(*refs):
    o_ref = refs[-1]
    w_ref, b_ref = refs[-3], refs[-2]
    s = refs[0][0] + refs[0][1]
    for p in refs[1:-3]:
        s = s + p[0] + p[1]
    y = jnp.dot(s, w_ref[...], preferred_element_type=jnp.float32)
    o_ref[...] = jnp.maximum(y + b_ref[...], 0.0)


def _mm_softmax_body(n_out, *refs):
    o_ref = refs[-1]
    w_ref, b_ref = refs[-3], refs[-2]
    s = refs[0][0] + refs[0][1]
    for p in refs[1:-3]:
        s = s + p[0] + p[1]
    z = jnp.dot(s, w_ref[...], preferred_element_type=jnp.float32)
    z = z + b_ref[...]
    m = jnp.max(z, axis=-1, keepdims=True)
    e = jnp.exp(z - m)
    p = e / jnp.sum(e, axis=-1, keepdims=True)
    o_ref[...] = p[:, :n_out]


def _tc_dense(body, parts, w, b, n_rows, n_cols, bm):
    grid = (n_rows // bm,)
    in_specs = ([pl.BlockSpec((NC, bm, D), lambda i: (0, i, 0))
                 for _ in parts]
                + [pl.BlockSpec((D, D), lambda i: (0, 0)),
                   pl.BlockSpec((1, D), lambda i: (0, 0))])
    return pl.pallas_call(
        body,
        grid=grid,
        in_specs=in_specs,
        out_specs=pl.BlockSpec((bm, n_cols), lambda i: (i, 0)),
        out_shape=jax.ShapeDtypeStruct((n_rows, n_cols), jnp.float32),
    )(*parts, w, b)


def kernel(x, edge_index, W0, b0, W1, b1, a_mu_0, a_log_sigma_0,
           a_mu_1, a_log_sigma_1):
    src = edge_index[0]
    dst = edge_index[1]

    K0 = (1832780943, 270669613)
    K1 = (64467757, 2916123636)

    rt2 = jnp.float32(1.4142135381698608)
    scale0 = jnp.stack([a_mu_0, jnp.exp(a_log_sigma_0) * rt2])
    scale1 = jnp.stack([a_mu_1, jnp.exp(a_log_sigma_1) * rt2])

    parts0 = []
    row = 0
    for epw, ch in L1_PARTS:
        nrows = epw * NW
        a_part = _rng_scale(K0[0], K0[1], scale0, row, nrows)
        parts0.append(_sc_segment(x, src, dst, a_part, epw, ch, row))
        row += nrows

    a1_parts = []
    row = 0
    for epw, ch in L2_PARTS:
        nrows = epw * NW
        a1_parts.append(_rng_scale(K1[0], K1[1], scale1, row, nrows))
        row += nrows

    h = _tc_dense(_mm_relu_body, parts0, W0, b0.reshape(1, D),
                  N_ACC, D, 640)

    parts1 = []
    row = 0
    for (epw, ch), a_part in zip(L2_PARTS, a1_parts):
        parts1.append(_sc_segment(h, src, dst, a_part, epw, ch, row))
        row += epw * NW

    n_out = W1.shape[1]
    W1p = jnp.zeros((D, D), jnp.float32).at[:, :n_out].set(W1)
    b1p = jnp.full((1, D), -1e30, jnp.float32).at[0, :n_out].set(b1)
    body = functools.partial(_mm_softmax_body, n_out)
    return _tc_dense(body, parts1, W1p, b1p, N_NODES, n_out, 400)
